# baseline (speedup 1.0000x reference)
import functools

import jax
import jax.numpy as jnp
from jax import lax
from jax.experimental import pallas as pl
from jax.experimental.pallas import tpu as pltpu
from jax.experimental.pallas import tpu_sc as plsc

_B, _N, _M = 4, 8192, 2048
_C1, _C2 = 64, 128
_C_OUT = 128
_TILE = 512
_BN = _B * _N
_BIGF = 3.0e38

_NC, _NS = 2, 16
_NW = _NC * _NS
_QW = _N // _NW
_S = 128

_TILE_C = 512
_TILE_D = 2048


def _knn_body(u_ref, k_ref, c_ref, idx_ref, w_ref):
    u = u_ref[...]
    kn = k_ref[...]

    k2row = jnp.sum(kn * kn, axis=0, keepdims=True)
    u2 = 0.25 * jnp.sum(u * u, axis=1, keepdims=True) - 0.25
    cross2 = jnp.dot(u, kn, preferred_element_type=jnp.float32)
    e2 = jnp.maximum(u2 + (cross2 + k2row), 0.0)

    del c_ref
    colf = jax.lax.broadcasted_iota(
        jnp.int32, (_TILE, _M), 1).astype(jnp.float32)
    bigf = jnp.float32(_BIGF)

    def extract(cur):
        v = jnp.min(cur, axis=1, keepdims=True)
        s = jnp.where(cur == v, colf, bigf)
        i = jnp.min(s, axis=1, keepdims=True)
        return v, i, jnp.where(s == i, bigf, cur)

    v1, i1, cur = extract(e2)
    v2, i2, cur = extract(cur)
    v3, i3, _ = extract(cur)

    r1 = 1.0 / (v1 + 1e-8)
    r2 = 1.0 / (v2 + 1e-8)
    r3 = 1.0 / (v3 + 1e-8)
    norm = r1 + r2 + r3

    idx_ref[...] = jnp.concatenate([i1, i2, i3], axis=1).astype(jnp.int32)
    w_ref[...] = jnp.concatenate(
        [r1 / norm, r2 / norm, r3 / norm], axis=1)


def _sc_gather_body(idx_hbm, tab_hbm, out_hbm, idx_v, rows_v, sem):
    wid = lax.axis_index("s") * _NC + lax.axis_index("c")
    base = wid * _QW

    def chunk(ci, carry):
        qb = base + ci * _S
        for k in range(3):
            pltpu.sync_copy(idx_hbm.at[pl.ds(k * _N + qb, _S)], idx_v)
            pltpu.async_copy(tab_hbm.at[idx_v], rows_v, sem).wait()
            pltpu.sync_copy(rows_v, out_hbm.at[pl.ds(k * _N + qb, _S)])
        return carry

    lax.fori_loop(0, _QW // _S, chunk, 0)


_sc_gather = functools.partial(
    pl.kernel,
    out_type=jax.ShapeDtypeStruct((3 * _N, _C2), jnp.float32),
    mesh=plsc.VectorSubcoreMesh(core_axis_name="c", subcore_axis_name="s"),
    scratch_types=[
        pltpu.VMEM((_S,), jnp.int32),
        pltpu.VMEM((_S, _C2), jnp.float32),
        pltpu.SemaphoreType.DMA,
    ],
)(_sc_gather_body)


def _mlp_body(r0_ref, r1_ref, r2_ref, w_ref, uf_ref, wa_ref, wb_ref,
              y_ref, acc_ref):
    w = w_ref[...]
    interp = (w[:, 0:1] * r0_ref[...] + w[:, 1:2] * r1_ref[...]
              + w[:, 2:3] * r2_ref[...])
    y = (jnp.dot(interp, wa_ref[...], preferred_element_type=jnp.float32)
         + jnp.dot(uf_ref[...], wb_ref[...],
                   preferred_element_type=jnp.float32))
    y_ref[...] = y

    s = jnp.sum(y, axis=0, keepdims=True)
    s2 = jnp.sum(y * y, axis=0, keepdims=True)
    rowi = jax.lax.broadcasted_iota(jnp.int32, (8, _C_OUT), 0)
    contrib = (jnp.where(rowi == 0, jnp.broadcast_to(s, (8, _C_OUT)), 0.0)
               + jnp.where(rowi == 1, jnp.broadcast_to(s2, (8, _C_OUT)), 0.0))

    @pl.when(pl.program_id(0) == 0)
    def _():
        acc_ref[...] = jnp.zeros_like(acc_ref)

    acc_ref[...] += contrib


def _bn_body(y_ref, acc_ref, gamma_ref, beta_ref, out_ref):
    s = acc_ref[0:1, :]
    s2 = acc_ref[1:2, :]
    cnt = jnp.float32(_BN)
    mean = s / cnt
    var = s2 / cnt - mean * mean
    scale = gamma_ref[...] * jax.lax.rsqrt(var + 1e-5)
    shift = beta_ref[...] - mean * scale
    out_ref[...] = jnp.maximum(y_ref[...] * scale + shift, 0.0)


def kernel(unknown, known, unknow_feats, known_feats, W0, gamma0, beta0):
    ones_u = jnp.ones((_B, _N, 1), jnp.float32)
    pad_u = jnp.zeros((_B, _N, 4), jnp.float32)
    ub = jnp.concatenate([-2.0 * unknown, ones_u, pad_u], axis=2)
    kt = jnp.transpose(known, (0, 2, 1))
    pad_k = jnp.zeros((_B, 5, _M), jnp.float32)
    kb = jnp.concatenate([kt, pad_k], axis=1)
    kf_t = jnp.transpose(known_feats, (0, 2, 1))
    uf_t = jnp.transpose(unknow_feats, (0, 2, 1))
    wa = jnp.transpose(W0[:, :_C2])
    wb = jnp.transpose(W0[:, _C2:])
    colf_row = jnp.broadcast_to(
        jnp.arange(_M, dtype=jnp.float32)[None, :], (8, _M))

    nb = _N // _TILE_C
    rows_b, w_b, ybn_b, acc_b = [], [], [], []
    for b in range(_B):
        idx3, w3 = pl.pallas_call(
            _knn_body,
            grid=(_N // _TILE,),
            in_specs=[
                pl.BlockSpec((_TILE, 8), lambda t: (t, 0)),
                pl.BlockSpec((8, _M), lambda t: (0, 0)),
                pl.BlockSpec((8, _M), lambda t: (0, 0)),
            ],
            out_specs=[
                pl.BlockSpec((_TILE, 3), lambda t: (t, 0)),
                pl.BlockSpec((_TILE, 3), lambda t: (t, 0)),
            ],
            out_shape=[
                jax.ShapeDtypeStruct((_N, 3), jnp.int32),
                jax.ShapeDtypeStruct((_N, 3), jnp.float32),
            ],
        )(ub[b], kb[b], colf_row)

        idx_planes = jnp.transpose(idx3).reshape(3 * _N)
        rows_b.append(_sc_gather(idx_planes, kf_t[b]))
        w_b.append(w3)

    for b in range(_B):
        ybn, acc = pl.pallas_call(
            _mlp_body,
            grid=(nb,),
            in_specs=[
                pl.BlockSpec((_TILE_C, _C2), lambda t: (t, 0)),
                pl.BlockSpec((_TILE_C, _C2), lambda t: (t + nb, 0)),
                pl.BlockSpec((_TILE_C, _C2), lambda t: (t + 2 * nb, 0)),
                pl.BlockSpec((_TILE_C, 3), lambda t: (t, 0)),
                pl.BlockSpec((_TILE_C, _C1), lambda t: (t, 0)),
                pl.BlockSpec((_C2, _C_OUT), lambda t: (0, 0)),
                pl.BlockSpec((_C1, _C_OUT), lambda t: (0, 0)),
            ],
            out_specs=[
                pl.BlockSpec((_TILE_C, _C_OUT), lambda t: (t, 0)),
                pl.BlockSpec((8, _C_OUT), lambda t: (0, 0)),
            ],
            out_shape=[
                jax.ShapeDtypeStruct((_N, _C_OUT), jnp.float32),
                jax.ShapeDtypeStruct((8, _C_OUT), jnp.float32),
            ],
        )(rows_b[b], rows_b[b], rows_b[b], w_b[b], uf_t[b], wa, wb)
        ybn_b.append(ybn)
        acc_b.append(acc)

    acc = acc_b[0] + acc_b[1] + acc_b[2] + acc_b[3]
    g2 = gamma0.reshape(1, _C_OUT)
    b2 = beta0.reshape(1, _C_OUT)

    out_b = []
    for b in range(_B):
        out_b.append(pl.pallas_call(
            _bn_body,
            grid=(_N // _TILE_D,),
            in_specs=[
                pl.BlockSpec((_TILE_D, _C_OUT), lambda t: (t, 0)),
                pl.BlockSpec((8, _C_OUT), lambda t: (0, 0)),
                pl.BlockSpec((1, _C_OUT), lambda t: (0, 0)),
                pl.BlockSpec((1, _C_OUT), lambda t: (0, 0)),
            ],
            out_specs=pl.BlockSpec((_TILE_D, _C_OUT), lambda t: (t, 0)),
            out_shape=jax.ShapeDtypeStruct((_N, _C_OUT), jnp.float32),
        )(ybn_b[b], acc, g2, b2))

    out = jnp.stack(out_b)
    return jnp.transpose(out, (0, 2, 1))


# speedup vs baseline: 1.0445x; 1.0445x over previous
import functools

import jax
import jax.numpy as jnp
from jax import lax
from jax.experimental import pallas as pl
from jax.experimental.pallas import tpu as pltpu
from jax.experimental.pallas import tpu_sc as plsc

_B, _N, _M = 4, 8192, 2048
_C1, _C2 = 64, 128
_C_OUT = 128
_TILE = 512
_BN = _B * _N
_BIGF = 3.0e38

_NC, _NS = 2, 16
_NW = _NC * _NS
_QW = _N // _NW
_S = 128

_TILE_C = 512
_TILE_D = 2048


def _knn_body(u_ref, k_ref, idx_ref, w_ref):
    u = u_ref[...]
    kn = k_ref[...]

    k2row = jnp.sum(kn * kn, axis=0, keepdims=True)
    u2 = 0.25 * jnp.sum(u * u, axis=1, keepdims=True) - 0.25
    cross2 = jnp.dot(u, kn, preferred_element_type=jnp.float32)
    e2 = jnp.maximum(u2 + (cross2 + k2row), 0.0)

    colf = jax.lax.broadcasted_iota(
        jnp.int32, (_TILE, _M), 1).astype(jnp.float32)
    bigf = jnp.float32(_BIGF)

    def extract(cur):
        v = jnp.min(cur, axis=1, keepdims=True)
        i = jnp.min(jnp.where(cur == v, colf, bigf), axis=1, keepdims=True)
        return v, i, colf == i

    v1, i1, eq1 = extract(e2)
    cur = jnp.where(eq1, bigf, e2)
    v2, i2, eq2 = extract(cur)
    cur = jnp.where(eq2, bigf, cur)
    v3, i3, _ = extract(cur)

    r1 = 1.0 / (v1 + 1e-8)
    r2 = 1.0 / (v2 + 1e-8)
    r3 = 1.0 / (v3 + 1e-8)
    norm = r1 + r2 + r3

    idx_ref[...] = jnp.concatenate([i1, i2, i3], axis=1).astype(jnp.int32)
    w_ref[...] = jnp.concatenate(
        [r1 / norm, r2 / norm, r3 / norm], axis=1)


def _sc_gather_body(idx_hbm, tab_hbm, out_hbm, idx_v, rows_v, sem):
    wid = lax.axis_index("s") * _NC + lax.axis_index("c")
    base = wid * _QW

    def chunk(ci, carry):
        qb = base + ci * _S
        for k in range(3):
            pltpu.sync_copy(idx_hbm.at[pl.ds(k * _N + qb, _S)], idx_v)
            pltpu.async_copy(tab_hbm.at[idx_v], rows_v, sem).wait()
            pltpu.sync_copy(rows_v, out_hbm.at[pl.ds(k * _N + qb, _S)])
        return carry

    lax.fori_loop(0, _QW // _S, chunk, 0)


_sc_gather = functools.partial(
    pl.kernel,
    out_type=jax.ShapeDtypeStruct((3 * _N, _C2), jnp.float32),
    mesh=plsc.VectorSubcoreMesh(core_axis_name="c", subcore_axis_name="s"),
    scratch_types=[
        pltpu.VMEM((_S,), jnp.int32),
        pltpu.VMEM((_S, _C2), jnp.float32),
        pltpu.SemaphoreType.DMA,
    ],
)(_sc_gather_body)


def _mlp_body(r0_ref, r1_ref, r2_ref, w_ref, uf_ref, wa_ref, wb_ref,
              y_ref, acc_ref):
    w = w_ref[...]
    interp = (w[:, 0:1] * r0_ref[...] + w[:, 1:2] * r1_ref[...]
              + w[:, 2:3] * r2_ref[...])
    y = (jnp.dot(interp, wa_ref[...], preferred_element_type=jnp.float32)
         + jnp.dot(uf_ref[...], wb_ref[...],
                   preferred_element_type=jnp.float32))
    y_ref[...] = y

    s = jnp.sum(y, axis=0, keepdims=True)
    s2 = jnp.sum(y * y, axis=0, keepdims=True)
    rowi = jax.lax.broadcasted_iota(jnp.int32, (8, _C_OUT), 0)
    contrib = (jnp.where(rowi == 0, jnp.broadcast_to(s, (8, _C_OUT)), 0.0)
               + jnp.where(rowi == 1, jnp.broadcast_to(s2, (8, _C_OUT)), 0.0))

    @pl.when(pl.program_id(0) == 0)
    def _():
        acc_ref[...] = jnp.zeros_like(acc_ref)

    acc_ref[...] += contrib


def _bn_body(y_ref, acc_ref, gamma_ref, beta_ref, out_ref):
    s = acc_ref[0:1, :]
    s2 = acc_ref[1:2, :]
    cnt = jnp.float32(_BN)
    mean = s / cnt
    var = s2 / cnt - mean * mean
    scale = gamma_ref[...] * jax.lax.rsqrt(var + 1e-5)
    shift = beta_ref[...] - mean * scale
    out_ref[...] = jnp.maximum(y_ref[...] * scale + shift, 0.0)


def kernel(unknown, known, unknow_feats, known_feats, W0, gamma0, beta0):
    ones_u = jnp.ones((_B, _N, 1), jnp.float32)
    pad_u = jnp.zeros((_B, _N, 4), jnp.float32)
    ub = jnp.concatenate([-2.0 * unknown, ones_u, pad_u], axis=2)
    kt = jnp.transpose(known, (0, 2, 1))
    pad_k = jnp.zeros((_B, 5, _M), jnp.float32)
    kb = jnp.concatenate([kt, pad_k], axis=1)
    kf_t = jnp.transpose(known_feats, (0, 2, 1))
    uf_t = jnp.transpose(unknow_feats, (0, 2, 1))
    wa = jnp.transpose(W0[:, :_C2])
    wb = jnp.transpose(W0[:, _C2:])

    nb = _N // _TILE_C
    rows_b, w_b, ybn_b, acc_b = [], [], [], []
    for b in range(_B):
        idx3, w3 = pl.pallas_call(
            _knn_body,
            grid=(_N // _TILE,),
            in_specs=[
                pl.BlockSpec((_TILE, 8), lambda t: (t, 0)),
                pl.BlockSpec((8, _M), lambda t: (0, 0)),
            ],
            out_specs=[
                pl.BlockSpec((_TILE, 3), lambda t: (t, 0)),
                pl.BlockSpec((_TILE, 3), lambda t: (t, 0)),
            ],
            out_shape=[
                jax.ShapeDtypeStruct((_N, 3), jnp.int32),
                jax.ShapeDtypeStruct((_N, 3), jnp.float32),
            ],
        )(ub[b], kb[b])

        idx_planes = jnp.transpose(idx3).reshape(3 * _N)
        rows_b.append(_sc_gather(idx_planes, kf_t[b]))
        w_b.append(w3)

    for b in range(_B):
        ybn, acc = pl.pallas_call(
            _mlp_body,
            grid=(nb,),
            in_specs=[
                pl.BlockSpec((_TILE_C, _C2), lambda t: (t, 0)),
                pl.BlockSpec((_TILE_C, _C2), lambda t: (t + nb, 0)),
                pl.BlockSpec((_TILE_C, _C2), lambda t: (t + 2 * nb, 0)),
                pl.BlockSpec((_TILE_C, 3), lambda t: (t, 0)),
                pl.BlockSpec((_TILE_C, _C1), lambda t: (t, 0)),
                pl.BlockSpec((_C2, _C_OUT), lambda t: (0, 0)),
                pl.BlockSpec((_C1, _C_OUT), lambda t: (0, 0)),
            ],
            out_specs=[
                pl.BlockSpec((_TILE_C, _C_OUT), lambda t: (t, 0)),
                pl.BlockSpec((8, _C_OUT), lambda t: (0, 0)),
            ],
            out_shape=[
                jax.ShapeDtypeStruct((_N, _C_OUT), jnp.float32),
                jax.ShapeDtypeStruct((8, _C_OUT), jnp.float32),
            ],
        )(rows_b[b], rows_b[b], rows_b[b], w_b[b], uf_t[b], wa, wb)
        ybn_b.append(ybn)
        acc_b.append(acc)

    acc = acc_b[0] + acc_b[1] + acc_b[2] + acc_b[3]
    g2 = gamma0.reshape(1, _C_OUT)
    b2 = beta0.reshape(1, _C_OUT)

    out_b = []
    for b in range(_B):
        out_b.append(pl.pallas_call(
            _bn_body,
            grid=(_N // _TILE_D,),
            in_specs=[
                pl.BlockSpec((_TILE_D, _C_OUT), lambda t: (t, 0)),
                pl.BlockSpec((8, _C_OUT), lambda t: (0, 0)),
                pl.BlockSpec((1, _C_OUT), lambda t: (0, 0)),
                pl.BlockSpec((1, _C_OUT), lambda t: (0, 0)),
            ],
            out_specs=pl.BlockSpec((_TILE_D, _C_OUT), lambda t: (t, 0)),
            out_shape=jax.ShapeDtypeStruct((_N, _C_OUT), jnp.float32),
        )(ybn_b[b], acc, g2, b2))

    out = jnp.stack(out_b)
    return jnp.transpose(out, (0, 2, 1))


# speedup vs baseline: 1.0471x; 1.0024x over previous
import functools

import jax
import jax.numpy as jnp
from jax import lax
from jax.experimental import pallas as pl
from jax.experimental.pallas import tpu as pltpu
from jax.experimental.pallas import tpu_sc as plsc

_B, _N, _M = 4, 8192, 2048
_C1, _C2 = 64, 128
_C_OUT = 128
_TILE = 1024
_BN = _B * _N
_BIGF = 3.0e38

_NC, _NS = 2, 16
_NW = _NC * _NS
_QW = _N // _NW
_S = 128

_TILE_C = 512
_TILE_D = 2048


def _knn_body(u_ref, k_ref, idx_ref, w_ref):
    u = u_ref[...]
    kn = k_ref[...]

    k2row = jnp.sum(kn * kn, axis=0, keepdims=True)
    u2 = 0.25 * jnp.sum(u * u, axis=1, keepdims=True) - 0.25
    cross2 = jnp.dot(u, kn, preferred_element_type=jnp.float32)
    e2 = jnp.maximum(u2 + (cross2 + k2row), 0.0)

    colf = jax.lax.broadcasted_iota(
        jnp.int32, (_TILE, _M), 1).astype(jnp.float32)
    bigf = jnp.float32(_BIGF)

    def extract(cur):
        v = jnp.min(cur, axis=1, keepdims=True)
        i = jnp.min(jnp.where(cur == v, colf, bigf), axis=1, keepdims=True)
        return v, i, colf == i

    v1, i1, eq1 = extract(e2)
    cur = jnp.where(eq1, bigf, e2)
    v2, i2, eq2 = extract(cur)
    cur = jnp.where(eq2, bigf, cur)
    v3, i3, _ = extract(cur)

    r1 = 1.0 / (v1 + 1e-8)
    r2 = 1.0 / (v2 + 1e-8)
    r3 = 1.0 / (v3 + 1e-8)
    norm = r1 + r2 + r3

    idx_ref[...] = jnp.concatenate([i1, i2, i3], axis=1).astype(jnp.int32)
    w_ref[...] = jnp.concatenate(
        [r1 / norm, r2 / norm, r3 / norm], axis=1)


def _sc_gather_body(idx_hbm, tab_hbm, out_hbm, idx_v, rows_v, sem):
    wid = lax.axis_index("s") * _NC + lax.axis_index("c")
    base = wid * _QW

    def chunk(ci, carry):
        qb = base + ci * _S
        for k in range(3):
            pltpu.sync_copy(idx_hbm.at[pl.ds(k * _N + qb, _S)], idx_v)
            pltpu.async_copy(tab_hbm.at[idx_v], rows_v, sem).wait()
            pltpu.sync_copy(rows_v, out_hbm.at[pl.ds(k * _N + qb, _S)])
        return carry

    lax.fori_loop(0, _QW // _S, chunk, 0)


_sc_gather = functools.partial(
    pl.kernel,
    out_type=jax.ShapeDtypeStruct((3 * _N, _C2), jnp.float32),
    mesh=plsc.VectorSubcoreMesh(core_axis_name="c", subcore_axis_name="s"),
    scratch_types=[
        pltpu.VMEM((_S,), jnp.int32),
        pltpu.VMEM((_S, _C2), jnp.float32),
        pltpu.SemaphoreType.DMA,
    ],
)(_sc_gather_body)


def _mlp_body(r0_ref, r1_ref, r2_ref, w_ref, uf_ref, wa_ref, wb_ref,
              y_ref, acc_ref):
    w = w_ref[...]
    interp = (w[:, 0:1] * r0_ref[...] + w[:, 1:2] * r1_ref[...]
              + w[:, 2:3] * r2_ref[...])
    y = (jnp.dot(interp, wa_ref[...], preferred_element_type=jnp.float32)
         + jnp.dot(uf_ref[...], wb_ref[...],
                   preferred_element_type=jnp.float32))
    y_ref[...] = y

    s = jnp.sum(y, axis=0, keepdims=True)
    s2 = jnp.sum(y * y, axis=0, keepdims=True)
    rowi = jax.lax.broadcasted_iota(jnp.int32, (8, _C_OUT), 0)
    contrib = (jnp.where(rowi == 0, jnp.broadcast_to(s, (8, _C_OUT)), 0.0)
               + jnp.where(rowi == 1, jnp.broadcast_to(s2, (8, _C_OUT)), 0.0))

    @pl.when(pl.program_id(0) == 0)
    def _():
        acc_ref[...] = jnp.zeros_like(acc_ref)

    acc_ref[...] += contrib


def _bn_body(y_ref, acc_ref, gamma_ref, beta_ref, out_ref):
    s = acc_ref[0:1, :]
    s2 = acc_ref[1:2, :]
    cnt = jnp.float32(_BN)
    mean = s / cnt
    var = s2 / cnt - mean * mean
    scale = gamma_ref[...] * jax.lax.rsqrt(var + 1e-5)
    shift = beta_ref[...] - mean * scale
    out_ref[...] = jnp.maximum(y_ref[...] * scale + shift, 0.0)


def kernel(unknown, known, unknow_feats, known_feats, W0, gamma0, beta0):
    ones_u = jnp.ones((_B, _N, 1), jnp.float32)
    pad_u = jnp.zeros((_B, _N, 4), jnp.float32)
    ub = jnp.concatenate([-2.0 * unknown, ones_u, pad_u], axis=2)
    kt = jnp.transpose(known, (0, 2, 1))
    pad_k = jnp.zeros((_B, 5, _M), jnp.float32)
    kb = jnp.concatenate([kt, pad_k], axis=1)
    kf_t = jnp.transpose(known_feats, (0, 2, 1))
    uf_t = jnp.transpose(unknow_feats, (0, 2, 1))
    wa = jnp.transpose(W0[:, :_C2])
    wb = jnp.transpose(W0[:, _C2:])

    nb = _N // _TILE_C
    rows_b, w_b, ybn_b, acc_b = [], [], [], []
    for b in range(_B):
        idx3, w3 = pl.pallas_call(
            _knn_body,
            grid=(_N // _TILE,),
            in_specs=[
                pl.BlockSpec((_TILE, 8), lambda t: (t, 0)),
                pl.BlockSpec((8, _M), lambda t: (0, 0)),
            ],
            out_specs=[
                pl.BlockSpec((_TILE, 3), lambda t: (t, 0)),
                pl.BlockSpec((_TILE, 3), lambda t: (t, 0)),
            ],
            out_shape=[
                jax.ShapeDtypeStruct((_N, 3), jnp.int32),
                jax.ShapeDtypeStruct((_N, 3), jnp.float32),
            ],
        )(ub[b], kb[b])

        idx_planes = jnp.transpose(idx3).reshape(3 * _N)
        rows_b.append(_sc_gather(idx_planes, kf_t[b]))
        w_b.append(w3)

    for b in range(_B):
        ybn, acc = pl.pallas_call(
            _mlp_body,
            grid=(nb,),
            in_specs=[
                pl.BlockSpec((_TILE_C, _C2), lambda t: (t, 0)),
                pl.BlockSpec((_TILE_C, _C2), lambda t: (t + nb, 0)),
                pl.BlockSpec((_TILE_C, _C2), lambda t: (t + 2 * nb, 0)),
                pl.BlockSpec((_TILE_C, 3), lambda t: (t, 0)),
                pl.BlockSpec((_TILE_C, _C1), lambda t: (t, 0)),
                pl.BlockSpec((_C2, _C_OUT), lambda t: (0, 0)),
                pl.BlockSpec((_C1, _C_OUT), lambda t: (0, 0)),
            ],
            out_specs=[
                pl.BlockSpec((_TILE_C, _C_OUT), lambda t: (t, 0)),
                pl.BlockSpec((8, _C_OUT), lambda t: (0, 0)),
            ],
            out_shape=[
                jax.ShapeDtypeStruct((_N, _C_OUT), jnp.float32),
                jax.ShapeDtypeStruct((8, _C_OUT), jnp.float32),
            ],
        )(rows_b[b], rows_b[b], rows_b[b], w_b[b], uf_t[b], wa, wb)
        ybn_b.append(ybn)
        acc_b.append(acc)

    acc = acc_b[0] + acc_b[1] + acc_b[2] + acc_b[3]
    g2 = gamma0.reshape(1, _C_OUT)
    b2 = beta0.reshape(1, _C_OUT)

    out_b = []
    for b in range(_B):
        out_b.append(pl.pallas_call(
            _bn_body,
            grid=(_N // _TILE_D,),
            in_specs=[
                pl.BlockSpec((_TILE_D, _C_OUT), lambda t: (t, 0)),
                pl.BlockSpec((8, _C_OUT), lambda t: (0, 0)),
                pl.BlockSpec((1, _C_OUT), lambda t: (0, 0)),
                pl.BlockSpec((1, _C_OUT), lambda t: (0, 0)),
            ],
            out_specs=pl.BlockSpec((_TILE_D, _C_OUT), lambda t: (t, 0)),
            out_shape=jax.ShapeDtypeStruct((_N, _C_OUT), jnp.float32),
        )(ybn_b[b], acc, g2, b2))

    out = jnp.stack(out_b)
    return jnp.transpose(out, (0, 2, 1))
